# baseline (speedup 1.0000x reference)
import functools

import jax
import jax.numpy as jnp
from jax import lax
from jax.experimental import pallas as pl
from jax.experimental.pallas import tpu as pltpu
from jax.experimental.pallas import tpu_sc as plsc

N = 10000
D = 128
NNZ = 320000
NUM_HG = 3
E = NUM_HG * NNZ
CHUNK = 128
NC = 2
NS = 16
L = 16
CPT = 240
G = 24
NUM_CHUNKS = NC * NS * CPT
E_PAD = NUM_CHUNKS * CHUNK
NPAD = 10240
ROWS_PER_TILE = NPAD // NS
ZROWS = 128


def _proj_body(x_ref, w_ref, vals_ref, alpha_ref, h_ref, sv_ref):
    h_ref[...] = lax.dot_general(
        x_ref[...], w_ref[...], (((1,), (1,)), ((), ())),
        preferred_element_type=jnp.float32)
    a = jax.nn.sigmoid(alpha_ref[...])
    sv_ref[...] = vals_ref[...] * a


def _project_and_scale(x, w, vals3, alpha):
    return pl.pallas_call(
        _proj_body,
        grid=(10,),
        in_specs=[
            pl.BlockSpec((N // 10, D), lambda i: (i, 0)),
            pl.BlockSpec((D, D), lambda i: (0, 0)),
            pl.BlockSpec((NUM_HG, NNZ // 10), lambda i: (0, i)),
            pl.BlockSpec((NUM_HG, 1), lambda i: (0, 0)),
        ],
        out_specs=[
            pl.BlockSpec((N // 10, D), lambda i: (i, 0)),
            pl.BlockSpec((NUM_HG, NNZ // 10), lambda i: (0, i)),
        ],
        out_shape=[
            jax.ShapeDtypeStruct((N, D), jnp.float32),
            jax.ShapeDtypeStruct((NUM_HG, NNZ), jnp.float32),
        ],
    )(x, w, vals3, alpha)


_mesh = plsc.VectorSubcoreMesh(core_axis_name="c", subcore_axis_name="s")


@functools.partial(
    pl.kernel,
    out_type=jax.ShapeDtypeStruct((NC, NPAD, D), jnp.float32),
    mesh=_mesh,
    scratch_types=[
        pltpu.VMEM_SHARED((NPAD, D), jnp.float32),
        pltpu.VMEM((G, CHUNK), jnp.int32),
        pltpu.VMEM((G, CHUNK), jnp.int32),
        pltpu.VMEM((G, CHUNK), jnp.float32),
        pltpu.VMEM((CHUNK, D), jnp.float32),
        pltpu.VMEM((CHUNK, D), jnp.float32),
        pltpu.SemaphoreType.DMA,
        pltpu.SemaphoreType.DMA,
    ],
)
def _spmm_kernel(h_hbm, cols_hbm, rows_hbm, vals_hbm, out_hbm,
                 acc, cidx, ridx, valv, buf_a, buf_b, gsem_a, gsem_b):
    cid = lax.axis_index("c")
    sid = lax.axis_index("s")
    w = cid * NS + sid

    zeros16 = jnp.zeros((L,), jnp.float32)

    @pl.loop(0, ZROWS)
    def _zero_buf(i):
        for j in range(D // L):
            buf_a[i, pl.ds(j * L, L)] = zeros16

    @pl.loop(0, ROWS_PER_TILE // ZROWS)
    def _zero_acc(t):
        pltpu.sync_copy(buf_a, acc.at[pl.ds(sid * ROWS_PER_TILE + t * ZROWS, ZROWS)])

    plsc.subcore_barrier()

    def _scale_and_scatter(c, buf):
        @plsc.parallel_loop(0, CHUNK // L)
        def _grp(g):
            vv = valv[c, pl.ds(g * L, L)]
            for l in range(L):
                vs = jnp.broadcast_to(vv[l], (L,))
                e = g * L + l
                for j in range(D // L):
                    sl = pl.ds(j * L, L)
                    buf[e, sl] = buf[e, sl] * vs

        pltpu.sync_copy(buf, acc.at[ridx.at[c]], add=True)

    @pl.loop(0, CPT // G)
    def _batch(b):
        base = w * CPT + b * G
        pltpu.sync_copy(cols_hbm.at[pl.ds(base, G)], cidx)
        pltpu.sync_copy(rows_hbm.at[pl.ds(base, G)], ridx)
        pltpu.sync_copy(vals_hbm.at[pl.ds(base, G)], valv)

        pltpu.async_copy(h_hbm.at[cidx.at[0]], buf_a, gsem_a)

        @pl.loop(0, G // 2)
        def _chunk(i):
            c = i * 2
            pltpu.async_copy(h_hbm.at[cidx.at[c + 1]], buf_b, gsem_b)
            pltpu.make_async_copy(h_hbm.at[cidx.at[c]], buf_a, gsem_a).wait()
            _scale_and_scatter(c, buf_a)

            @pl.when(c + 2 < G)
            def _():
                pltpu.async_copy(h_hbm.at[cidx.at[c + 2]], buf_a, gsem_a)

            pltpu.make_async_copy(h_hbm.at[cidx.at[c + 1]], buf_b, gsem_b).wait()
            _scale_and_scatter(c + 1, buf_b)

    plsc.subcore_barrier()

    r0 = sid * ROWS_PER_TILE
    pltpu.sync_copy(acc.at[pl.ds(r0, ROWS_PER_TILE)],
                    out_hbm.at[cid].at[pl.ds(r0, ROWS_PER_TILE)])


def _combine_body(p_ref, o_ref):
    o_ref[...] = p_ref[0] + p_ref[1]


def _combine(partials):
    return pl.pallas_call(
        _combine_body,
        grid=(10,),
        in_specs=[pl.BlockSpec((NC, N // 10, D), lambda i: (0, i, 0))],
        out_specs=pl.BlockSpec((N // 10, D), lambda i: (i, 0)),
        out_shape=jax.ShapeDtypeStruct((N, D), jnp.float32),
    )(partials)


def kernel(company_emb, lap0_idx, lap0_val, lap1_idx, lap1_val, lap2_idx,
           lap2_val, W, alpha):
    vals3 = jnp.stack([lap0_val, lap1_val, lap2_val])
    h, vals_scaled = _project_and_scale(company_emb, W, vals3, alpha)
    pad = E_PAD - E
    cols = jnp.concatenate(
        [lap0_idx[1], lap1_idx[1], lap2_idx[1],
         jnp.zeros((pad,), lap0_idx.dtype)]).astype(jnp.int32)
    rows = jnp.concatenate(
        [lap0_idx[0], lap1_idx[0], lap2_idx[0],
         jnp.full((pad,), N, lap0_idx.dtype)]).astype(jnp.int32)
    vals = jnp.concatenate(
        [vals_scaled.reshape(-1), jnp.zeros((pad,), jnp.float32)])
    partials = _spmm_kernel(
        h,
        cols.reshape(NUM_CHUNKS, CHUNK),
        rows.reshape(NUM_CHUNKS, CHUNK),
        vals.reshape(NUM_CHUNKS, CHUNK),
    )
    return _combine(partials)


# speedup vs baseline: 2.2182x; 2.2182x over previous
import functools

import jax
import jax.numpy as jnp
from jax import lax
from jax.experimental import pallas as pl
from jax.experimental.pallas import tpu as pltpu
from jax.experimental.pallas import tpu_sc as plsc

N = 10000
D = 128
DH = D // 2
NNZ = 320000
NUM_HG = 3
E = NUM_HG * NNZ
CHUNK = 128
NC = 2
NS = 16
L = 16
CPT = 480
G = 16
NUM_CHUNKS = NS * CPT
E_PAD = NUM_CHUNKS * CHUNK
NPAD = 10240
ROWS_PER_TILE = NPAD // NS


def _proj_body(x_ref, w_ref, vals_ref, alpha_ref, h_ref, sv_ref):
    h = lax.dot_general(
        x_ref[...], w_ref[...], (((1,), (1,)), ((), ())),
        preferred_element_type=jnp.float32)
    h_ref[0] = h[:, :DH]
    h_ref[1] = h[:, DH:]
    a = jax.nn.sigmoid(alpha_ref[...])
    sv_ref[...] = vals_ref[...] * a


def _project_and_scale(x, w, vals3, alpha):
    return pl.pallas_call(
        _proj_body,
        grid=(10,),
        in_specs=[
            pl.BlockSpec((NPAD // 10, D), lambda i: (i, 0)),
            pl.BlockSpec((D, D), lambda i: (0, 0)),
            pl.BlockSpec((NUM_HG, NNZ // 10), lambda i: (0, i)),
            pl.BlockSpec((NUM_HG, 1), lambda i: (0, 0)),
        ],
        out_specs=[
            pl.BlockSpec((NC, NPAD // 10, DH), lambda i: (0, i, 0)),
            pl.BlockSpec((NUM_HG, NNZ // 10), lambda i: (0, i)),
        ],
        out_shape=[
            jax.ShapeDtypeStruct((NC, NPAD, DH), jnp.float32),
            jax.ShapeDtypeStruct((NUM_HG, NNZ), jnp.float32),
        ],
    )(x, w, vals3, alpha)


_mesh = plsc.VectorSubcoreMesh(core_axis_name="c", subcore_axis_name="s")


@functools.partial(
    pl.kernel,
    out_type=jax.ShapeDtypeStruct((NC, NPAD, DH), jnp.float32),
    mesh=_mesh,
    scratch_types=[
        pltpu.VMEM_SHARED((NPAD, DH), jnp.float32),
        pltpu.VMEM_SHARED((NPAD, DH), jnp.float32),
        pltpu.VMEM((G, CHUNK), jnp.int32),
        pltpu.VMEM((G, CHUNK), jnp.int32),
        pltpu.VMEM((G, CHUNK), jnp.float32),
        pltpu.VMEM((CHUNK, DH), jnp.float32),
        pltpu.VMEM((CHUNK, DH), jnp.float32),
        pltpu.SemaphoreType.DMA,
        pltpu.SemaphoreType.DMA,
    ],
)
def _spmm_kernel(h_hbm, cols_hbm, rows_hbm, vals_hbm, out_hbm,
                 acc, hs, cidx, ridx, valv, buf_a, buf_b, gsem_a, gsem_b):
    cid = lax.axis_index("c")
    sid = lax.axis_index("s")

    r0 = sid * ROWS_PER_TILE

    pltpu.sync_copy(h_hbm.at[cid].at[pl.ds(r0, ROWS_PER_TILE)],
                    hs.at[pl.ds(r0, ROWS_PER_TILE)])

    zeros16 = jnp.zeros((L,), jnp.float32)

    @pl.loop(0, CHUNK)
    def _zero_buf(i):
        for j in range(DH // L):
            buf_a[i, pl.ds(j * L, L)] = zeros16

    @pl.loop(0, ROWS_PER_TILE // CHUNK)
    def _zero_acc(t):
        pltpu.sync_copy(buf_a, acc.at[pl.ds(r0 + t * CHUNK, CHUNK)])

    plsc.subcore_barrier()

    def _scale_and_scatter(c, buf):
        @plsc.parallel_loop(0, CHUNK // L)
        def _grp(g):
            vv = valv[c, pl.ds(g * L, L)]
            for l in range(L):
                vs = jnp.broadcast_to(vv[l], (L,))
                e = g * L + l
                for j in range(DH // L):
                    sl = pl.ds(j * L, L)
                    buf[e, sl] = buf[e, sl] * vs

        pltpu.sync_copy(buf, acc.at[ridx.at[c]], add=True)

    @pl.loop(0, CPT // G)
    def _batch(b):
        base = sid * CPT + b * G
        pltpu.sync_copy(cols_hbm.at[pl.ds(base, G)], cidx)
        pltpu.sync_copy(rows_hbm.at[pl.ds(base, G)], ridx)
        pltpu.sync_copy(vals_hbm.at[pl.ds(base, G)], valv)

        pltpu.async_copy(hs.at[cidx.at[0]], buf_a, gsem_a)

        @pl.loop(0, G // 2)
        def _chunk(i):
            c = i * 2
            pltpu.async_copy(hs.at[cidx.at[c + 1]], buf_b, gsem_b)
            pltpu.make_async_copy(hs.at[cidx.at[c]], buf_a, gsem_a).wait()
            _scale_and_scatter(c, buf_a)

            @pl.when(c + 2 < G)
            def _():
                pltpu.async_copy(hs.at[cidx.at[c + 2]], buf_a, gsem_a)

            pltpu.make_async_copy(hs.at[cidx.at[c + 1]], buf_b, gsem_b).wait()
            _scale_and_scatter(c + 1, buf_b)

    plsc.subcore_barrier()

    pltpu.sync_copy(acc.at[pl.ds(r0, ROWS_PER_TILE)],
                    out_hbm.at[cid].at[pl.ds(r0, ROWS_PER_TILE)])


def _combine_body(p_ref, o_ref):
    o_ref[:, :DH] = p_ref[0]
    o_ref[:, DH:] = p_ref[1]


def _combine(partials):
    return pl.pallas_call(
        _combine_body,
        grid=(10,),
        in_specs=[pl.BlockSpec((NC, N // 10, DH), lambda i: (0, i, 0))],
        out_specs=pl.BlockSpec((N // 10, D), lambda i: (i, 0)),
        out_shape=jax.ShapeDtypeStruct((N, D), jnp.float32),
    )(partials)


def kernel(company_emb, lap0_idx, lap0_val, lap1_idx, lap1_val, lap2_idx,
           lap2_val, W, alpha):
    vals3 = jnp.stack([lap0_val, lap1_val, lap2_val])
    x_pad = jnp.pad(company_emb, ((0, NPAD - N), (0, 0)))
    h2, vals_scaled = _project_and_scale(x_pad, W, vals3, alpha)
    pad = E_PAD - E
    cols = jnp.concatenate(
        [lap0_idx[1], lap1_idx[1], lap2_idx[1],
         jnp.zeros((pad,), lap0_idx.dtype)]).astype(jnp.int32)
    rows = jnp.concatenate(
        [lap0_idx[0], lap1_idx[0], lap2_idx[0],
         jnp.full((pad,), N, lap0_idx.dtype)]).astype(jnp.int32)
    vals = jnp.concatenate(
        [vals_scaled.reshape(-1), jnp.zeros((pad,), jnp.float32)])
    partials = _spmm_kernel(
        h2,
        cols.reshape(NUM_CHUNKS, CHUNK),
        rows.reshape(NUM_CHUNKS, CHUNK),
        vals.reshape(NUM_CHUNKS, CHUNK),
    )
    return _combine(partials)


# speedup vs baseline: 2.3120x; 1.0423x over previous
import functools

import jax
import jax.numpy as jnp
from jax import lax
from jax.experimental import pallas as pl
from jax.experimental.pallas import tpu as pltpu
from jax.experimental.pallas import tpu_sc as plsc

N = 10000
D = 128
DH = D // 2
NNZ = 320000
NUM_HG = 3
E = NUM_HG * NNZ
CHUNK = 128
NC = 2
NS = 16
L = 16
CPT = 480
G = 16
NUM_CHUNKS = NS * CPT
E_PAD = NUM_CHUNKS * CHUNK
NPAD = 10240
ROWS_PER_TILE = NPAD // NS


def _proj_body(x_ref, w_ref, vals_ref, alpha_ref, h_ref, sv_ref):
    h = lax.dot_general(
        x_ref[...], w_ref[...], (((1,), (1,)), ((), ())),
        preferred_element_type=jnp.float32)
    h_ref[0] = h[:, :DH]
    h_ref[1] = h[:, DH:]
    a = jax.nn.sigmoid(alpha_ref[...])
    sv_ref[...] = vals_ref[...] * a


def _project_and_scale(x, w, vals3, alpha):
    return pl.pallas_call(
        _proj_body,
        grid=(10,),
        in_specs=[
            pl.BlockSpec((NPAD // 10, D), lambda i: (i, 0)),
            pl.BlockSpec((D, D), lambda i: (0, 0)),
            pl.BlockSpec((NUM_HG, NNZ // 10), lambda i: (0, i)),
            pl.BlockSpec((NUM_HG, 1), lambda i: (0, 0)),
        ],
        out_specs=[
            pl.BlockSpec((NC, NPAD // 10, DH), lambda i: (0, i, 0)),
            pl.BlockSpec((NUM_HG, NNZ // 10), lambda i: (0, i)),
        ],
        out_shape=[
            jax.ShapeDtypeStruct((NC, NPAD, DH), jnp.float32),
            jax.ShapeDtypeStruct((NUM_HG, NNZ), jnp.float32),
        ],
    )(x, w, vals3, alpha)


_mesh = plsc.VectorSubcoreMesh(core_axis_name="c", subcore_axis_name="s")


@functools.partial(
    pl.kernel,
    out_type=jax.ShapeDtypeStruct((NC, NPAD, DH), jnp.float32),
    mesh=_mesh,
    scratch_types=[
        pltpu.VMEM_SHARED((NPAD, DH), jnp.float32),
        pltpu.VMEM_SHARED((NPAD, DH), jnp.float32),
        pltpu.VMEM((G, CHUNK), jnp.int32),
        pltpu.VMEM((G, CHUNK), jnp.int32),
        pltpu.VMEM((G, CHUNK), jnp.float32),
        pltpu.VMEM((CHUNK, DH), jnp.float32),
        pltpu.VMEM((CHUNK, DH), jnp.float32),
        pltpu.SemaphoreType.DMA,
        pltpu.SemaphoreType.DMA,
        pltpu.SemaphoreType.DMA,
        pltpu.SemaphoreType.DMA,
    ],
)
def _spmm_kernel(h_hbm, cols_hbm, rows_hbm, vals_hbm, out_hbm,
                 acc, hs, cidx, ridx, valv, buf_a, buf_b, gsem_a, gsem_b,
                 ssem_a, ssem_b):
    cid = lax.axis_index("c")
    sid = lax.axis_index("s")

    r0 = sid * ROWS_PER_TILE

    pltpu.sync_copy(h_hbm.at[cid].at[pl.ds(r0, ROWS_PER_TILE)],
                    hs.at[pl.ds(r0, ROWS_PER_TILE)])

    zeros16 = jnp.zeros((L,), jnp.float32)

    @pl.loop(0, CHUNK)
    def _zero_buf(i):
        for j in range(DH // L):
            buf_a[i, pl.ds(j * L, L)] = zeros16

    @pl.loop(0, ROWS_PER_TILE // CHUNK)
    def _zero_acc(t):
        pltpu.sync_copy(buf_a, acc.at[pl.ds(r0 + t * CHUNK, CHUNK)])

    plsc.subcore_barrier()

    def _scale(c, buf):
        @plsc.parallel_loop(0, CHUNK // L)
        def _grp(g):
            vv = valv[c, pl.ds(g * L, L)]
            for l in range(L):
                vs = jnp.broadcast_to(vv[l], (L,))
                e = g * L + l
                for j in range(DH // L):
                    sl = pl.ds(j * L, L)
                    buf[e, sl] = buf[e, sl] * vs

    @pl.loop(0, CPT // G)
    def _batch(b):
        base = sid * CPT + b * G
        pltpu.sync_copy(cols_hbm.at[pl.ds(base, G)], cidx)
        pltpu.sync_copy(rows_hbm.at[pl.ds(base, G)], ridx)
        pltpu.sync_copy(vals_hbm.at[pl.ds(base, G)], valv)

        pltpu.async_copy(hs.at[cidx.at[0]], buf_a, gsem_a)

        @pl.loop(0, G // 2)
        def _chunk(i):
            c = i * 2

            @pl.when(c > 0)
            def _():
                pltpu.make_async_copy(
                    buf_b, acc.at[ridx.at[c - 1]], ssem_b).wait()

            pltpu.async_copy(hs.at[cidx.at[c + 1]], buf_b, gsem_b)
            pltpu.make_async_copy(hs.at[cidx.at[c]], buf_a, gsem_a).wait()
            _scale(c, buf_a)
            pltpu.async_copy(buf_a, acc.at[ridx.at[c]], ssem_a, add=True)
            pltpu.make_async_copy(hs.at[cidx.at[c + 1]], buf_b, gsem_b).wait()
            _scale(c + 1, buf_b)

            @pl.when(c + 2 < G)
            def _():
                pltpu.make_async_copy(
                    buf_a, acc.at[ridx.at[c]], ssem_a).wait()
                pltpu.async_copy(hs.at[cidx.at[c + 2]], buf_a, gsem_a)

            pltpu.async_copy(buf_b, acc.at[ridx.at[c + 1]], ssem_b, add=True)

        pltpu.make_async_copy(buf_a, acc.at[ridx.at[G - 2]], ssem_a).wait()
        pltpu.make_async_copy(buf_b, acc.at[ridx.at[G - 1]], ssem_b).wait()

    plsc.subcore_barrier()

    pltpu.sync_copy(acc.at[pl.ds(r0, ROWS_PER_TILE)],
                    out_hbm.at[cid].at[pl.ds(r0, ROWS_PER_TILE)])


def _combine_body(p_ref, o_ref):
    o_ref[:, :DH] = p_ref[0]
    o_ref[:, DH:] = p_ref[1]


def _combine(partials):
    return pl.pallas_call(
        _combine_body,
        grid=(10,),
        in_specs=[pl.BlockSpec((NC, N // 10, DH), lambda i: (0, i, 0))],
        out_specs=pl.BlockSpec((N // 10, D), lambda i: (i, 0)),
        out_shape=jax.ShapeDtypeStruct((N, D), jnp.float32),
    )(partials)


def kernel(company_emb, lap0_idx, lap0_val, lap1_idx, lap1_val, lap2_idx,
           lap2_val, W, alpha):
    vals3 = jnp.stack([lap0_val, lap1_val, lap2_val])
    x_pad = jnp.pad(company_emb, ((0, NPAD - N), (0, 0)))
    h2, vals_scaled = _project_and_scale(x_pad, W, vals3, alpha)
    pad = E_PAD - E
    cols = jnp.concatenate(
        [lap0_idx[1], lap1_idx[1], lap2_idx[1],
         jnp.zeros((pad,), lap0_idx.dtype)]).astype(jnp.int32)
    rows = jnp.concatenate(
        [lap0_idx[0], lap1_idx[0], lap2_idx[0],
         jnp.full((pad,), N, lap0_idx.dtype)]).astype(jnp.int32)
    vals = jnp.concatenate(
        [vals_scaled.reshape(-1), jnp.zeros((pad,), jnp.float32)])
    partials = _spmm_kernel(
        h2,
        cols.reshape(NUM_CHUNKS, CHUNK),
        rows.reshape(NUM_CHUNKS, CHUNK),
        vals.reshape(NUM_CHUNKS, CHUNK),
    )
    return _combine(partials)


# speedup vs baseline: 2.3271x; 1.0065x over previous
import functools

import jax
import jax.numpy as jnp
from jax import lax
from jax.experimental import pallas as pl
from jax.experimental.pallas import tpu as pltpu
from jax.experimental.pallas import tpu_sc as plsc

N = 10000
D = 128
DH = D // 2
NNZ = 320000
NUM_HG = 3
E = NUM_HG * NNZ
CHUNK = 128
NC = 2
NS = 16
L = 16
CPT = 480
G = 16
NUM_CHUNKS = NS * CPT
E_PAD = NUM_CHUNKS * CHUNK
NPAD = 10240
ROWS_PER_TILE = NPAD // NS


def _proj_body(x_ref, w_ref, vals_ref, alpha_ref, h_ref, sv_ref):
    h = lax.dot_general(
        x_ref[...], w_ref[...], (((1,), (1,)), ((), ())),
        preferred_element_type=jnp.float32)
    h_ref[0] = h[:, :DH]
    h_ref[1] = h[:, DH:]
    a = jax.nn.sigmoid(alpha_ref[...])
    sv_ref[...] = vals_ref[...] * a


def _project_and_scale(x, w, vals3, alpha):
    return pl.pallas_call(
        _proj_body,
        grid=(10,),
        in_specs=[
            pl.BlockSpec((N // 10, D), lambda i: (i, 0)),
            pl.BlockSpec((D, D), lambda i: (0, 0)),
            pl.BlockSpec((NUM_HG, NNZ // 10), lambda i: (0, i)),
            pl.BlockSpec((NUM_HG, 1), lambda i: (0, 0)),
        ],
        out_specs=[
            pl.BlockSpec((NC, N // 10, DH), lambda i: (0, i, 0)),
            pl.BlockSpec((NUM_HG, NNZ // 10), lambda i: (0, i)),
        ],
        out_shape=[
            jax.ShapeDtypeStruct((NC, NPAD, DH), jnp.float32),
            jax.ShapeDtypeStruct((NUM_HG, NNZ), jnp.float32),
        ],
    )(x, w, vals3, alpha)


_mesh = plsc.VectorSubcoreMesh(core_axis_name="c", subcore_axis_name="s")


@functools.partial(
    pl.kernel,
    out_type=jax.ShapeDtypeStruct((NC, NPAD, DH), jnp.float32),
    mesh=_mesh,
    scratch_types=[
        pltpu.VMEM_SHARED((NPAD, DH), jnp.float32),
        pltpu.VMEM_SHARED((NPAD, DH), jnp.float32),
        pltpu.VMEM((G, CHUNK), jnp.int32),
        pltpu.VMEM((G, CHUNK), jnp.int32),
        pltpu.VMEM((G, CHUNK), jnp.float32),
        pltpu.VMEM((CHUNK, DH), jnp.float32),
        pltpu.VMEM((CHUNK, DH), jnp.float32),
        pltpu.SemaphoreType.DMA,
        pltpu.SemaphoreType.DMA,
        pltpu.SemaphoreType.DMA,
        pltpu.SemaphoreType.DMA,
    ],
)
def _spmm_kernel(h_hbm, cols_hbm, rows_hbm, vals_hbm, out_hbm,
                 acc, hs, cidx, ridx, valv, buf_a, buf_b, gsem_a, gsem_b,
                 ssem_a, ssem_b):
    cid = lax.axis_index("c")
    sid = lax.axis_index("s")

    r0 = sid * ROWS_PER_TILE

    pltpu.sync_copy(h_hbm.at[cid].at[pl.ds(r0, ROWS_PER_TILE)],
                    hs.at[pl.ds(r0, ROWS_PER_TILE)])

    zeros16 = jnp.zeros((L,), jnp.float32)

    @pl.loop(0, CHUNK)
    def _zero_buf(i):
        for j in range(DH // L):
            buf_a[i, pl.ds(j * L, L)] = zeros16

    @pl.loop(0, ROWS_PER_TILE // CHUNK)
    def _zero_acc(t):
        pltpu.sync_copy(buf_a, acc.at[pl.ds(r0 + t * CHUNK, CHUNK)])

    plsc.subcore_barrier()

    def _scale(c, buf):
        @plsc.parallel_loop(0, CHUNK // L)
        def _grp(g):
            vv = valv[c, pl.ds(g * L, L)]
            for l in range(L):
                vs = jnp.broadcast_to(vv[l], (L,))
                e = g * L + l
                for j in range(DH // L):
                    sl = pl.ds(j * L, L)
                    buf[e, sl] = buf[e, sl] * vs

    @pl.loop(0, CPT // G)
    def _batch(b):
        base = sid * CPT + b * G
        pltpu.sync_copy(cols_hbm.at[pl.ds(base, G)], cidx)
        pltpu.sync_copy(rows_hbm.at[pl.ds(base, G)], ridx)
        pltpu.sync_copy(vals_hbm.at[pl.ds(base, G)], valv)

        pltpu.async_copy(hs.at[cidx.at[0]], buf_a, gsem_a)

        @pl.loop(0, G // 2)
        def _chunk(i):
            c = i * 2

            @pl.when(c > 0)
            def _():
                pltpu.make_async_copy(
                    buf_b, acc.at[ridx.at[c - 1]], ssem_b).wait()

            pltpu.async_copy(hs.at[cidx.at[c + 1]], buf_b, gsem_b)
            pltpu.make_async_copy(hs.at[cidx.at[c]], buf_a, gsem_a).wait()
            _scale(c, buf_a)
            pltpu.async_copy(buf_a, acc.at[ridx.at[c]], ssem_a, add=True)
            pltpu.make_async_copy(hs.at[cidx.at[c + 1]], buf_b, gsem_b).wait()
            _scale(c + 1, buf_b)

            @pl.when(c + 2 < G)
            def _():
                pltpu.make_async_copy(
                    buf_a, acc.at[ridx.at[c]], ssem_a).wait()
                pltpu.async_copy(hs.at[cidx.at[c + 2]], buf_a, gsem_a)

            pltpu.async_copy(buf_b, acc.at[ridx.at[c + 1]], ssem_b, add=True)

        pltpu.make_async_copy(buf_a, acc.at[ridx.at[G - 2]], ssem_a).wait()
        pltpu.make_async_copy(buf_b, acc.at[ridx.at[G - 1]], ssem_b).wait()

    plsc.subcore_barrier()

    pltpu.sync_copy(acc.at[pl.ds(r0, ROWS_PER_TILE)],
                    out_hbm.at[cid].at[pl.ds(r0, ROWS_PER_TILE)])


def _combine_body(p_ref, o_ref):
    o_ref[:, :DH] = p_ref[0]
    o_ref[:, DH:] = p_ref[1]


def _combine(partials):
    return pl.pallas_call(
        _combine_body,
        grid=(10,),
        in_specs=[pl.BlockSpec((NC, N // 10, DH), lambda i: (0, i, 0))],
        out_specs=pl.BlockSpec((N // 10, D), lambda i: (i, 0)),
        out_shape=jax.ShapeDtypeStruct((N, D), jnp.float32),
    )(partials)


def kernel(company_emb, lap0_idx, lap0_val, lap1_idx, lap1_val, lap2_idx,
           lap2_val, W, alpha):
    vals3 = jnp.stack([lap0_val, lap1_val, lap2_val])
    h2, vals_scaled = _project_and_scale(company_emb, W, vals3, alpha)
    pad = E_PAD - E
    cols = jnp.concatenate(
        [lap0_idx[1], lap1_idx[1], lap2_idx[1],
         jnp.zeros((pad,), lap0_idx.dtype)]).astype(jnp.int32)
    rows = jnp.concatenate(
        [lap0_idx[0], lap1_idx[0], lap2_idx[0],
         jnp.full((pad,), N, lap0_idx.dtype)]).astype(jnp.int32)
    vals = jnp.concatenate(
        [vals_scaled.reshape(-1), jnp.zeros((pad,), jnp.float32)])
    partials = _spmm_kernel(
        h2,
        cols.reshape(NUM_CHUNKS, CHUNK),
        rows.reshape(NUM_CHUNKS, CHUNK),
        vals.reshape(NUM_CHUNKS, CHUNK),
    )
    return _combine(partials)
